# baseline (speedup 1.0000x reference)
import functools

import jax
import jax.numpy as jnp
from jax import lax
from jax.experimental import pallas as pl
from jax.experimental.pallas import tpu as pltpu
from jax.experimental.pallas import tpu_sc as plsc

N_FIELDS = 26
VOCAB = 100000
HIDDEN = 64
BATCH = 16384

NC = 2
NS = 16
L = 16
NW = NC * NS
NPAIR = N_FIELDS * HIDDEN
PPT = NPAIR // NW
BCHUNK = 4096
UNROLL = 64


def _body(x_hbm, tab_hbm, out_hbm, vocab_v, x_v, res0, res1, ssem, osem):
    cid = lax.axis_index("c")
    sid = lax.axis_index("s")
    wid = sid * NC + cid
    p0 = wid * PPT
    res = (res0, res1)

    def pair_body(r, prev_f):
        p = p0 + r
        f = p // HIDDEN
        h = p % HIDDEN

        pltpu.async_copy(tab_hbm.at[f, h], vocab_v, ssem)

        @pl.when(f != prev_f)
        def _():
            pltpu.sync_copy(x_hbm.at[f], x_v)

        pltpu.make_async_copy(tab_hbm.at[f, h], vocab_v, ssem).wait()

        for c in range(BATCH // BCHUNK):
            b = c % 2

            @pl.when((r > 0) | (c > 1))
            def _():
                pltpu.make_async_copy(
                    res[b], out_hbm.at[f, h, pl.ds(0, BCHUNK)], osem
                ).wait()

            @plsc.parallel_loop(0, BCHUNK, step=L, unroll=UNROLL)
            def _(o):
                idx = x_v[pl.ds(c * BCHUNK + o, L)]
                res[b][pl.ds(o, L)] = plsc.load_gather(vocab_v, [idx])
            pltpu.async_copy(
                res[b], out_hbm.at[f, h, pl.ds(c * BCHUNK, BCHUNK)], osem
            )
        return f

    last_f = lax.fori_loop(0, PPT, pair_body, -1)
    for _ in range(2):
        pltpu.make_async_copy(
            res0, out_hbm.at[last_f, 0, pl.ds(0, BCHUNK)], osem
        ).wait()


def kernel(x_cat, tables):
    x_t = x_cat.T
    tab_t = tables.transpose(0, 2, 1)
    mesh = plsc.VectorSubcoreMesh(core_axis_name="c", subcore_axis_name="s")
    run = functools.partial(
        pl.kernel,
        out_type=jax.ShapeDtypeStruct((N_FIELDS, HIDDEN, BATCH), jnp.float32),
        mesh=mesh,
        compiler_params=pltpu.CompilerParams(needs_layout_passes=False),
        scratch_types=[
            pltpu.VMEM((VOCAB,), jnp.float32),
            pltpu.VMEM((BATCH,), jnp.int32),
            pltpu.VMEM((BCHUNK,), jnp.float32),
            pltpu.VMEM((BCHUNK,), jnp.float32),
            pltpu.SemaphoreType.DMA,
            pltpu.SemaphoreType.DMA,
        ],
    )(_body)
    out_t = run(x_t, tab_t)
    return out_t.transpose(2, 0, 1)


# speedup vs baseline: 1.0027x; 1.0027x over previous
import functools

import jax
import jax.numpy as jnp
from jax import lax
from jax.experimental import pallas as pl
from jax.experimental.pallas import tpu as pltpu
from jax.experimental.pallas import tpu_sc as plsc

N_FIELDS = 26
VOCAB = 100000
HIDDEN = 64
BATCH = 16384

NC = 2
NS = 16
L = 16
NW = NC * NS
NPAIR = N_FIELDS * HIDDEN
PPT = NPAIR // NW
BCHUNK = 4096
UNROLL = 32


def _body(x_hbm, tab_hbm, out_hbm, vocab_v, x_v, res0, res1, ssem, osem):
    cid = lax.axis_index("c")
    sid = lax.axis_index("s")
    wid = sid * NC + cid
    p0 = wid * PPT
    res = (res0, res1)

    def pair_body(r, prev_f):
        p = p0 + r
        f = p // HIDDEN
        h = p % HIDDEN

        pltpu.async_copy(tab_hbm.at[f, h], vocab_v, ssem)

        @pl.when(f != prev_f)
        def _():
            pltpu.sync_copy(x_hbm.at[f], x_v)

        pltpu.make_async_copy(tab_hbm.at[f, h], vocab_v, ssem).wait()

        for c in range(BATCH // BCHUNK):
            b = c % 2

            @pl.when((r > 0) | (c > 1))
            def _():
                pltpu.make_async_copy(
                    res[b], out_hbm.at[f, h, pl.ds(0, BCHUNK)], osem
                ).wait()

            @plsc.parallel_loop(0, BCHUNK, step=L, unroll=UNROLL)
            def _(o):
                idx = x_v[pl.ds(c * BCHUNK + o, L)]
                res[b][pl.ds(o, L)] = plsc.load_gather(vocab_v, [idx])
            pltpu.async_copy(
                res[b], out_hbm.at[f, h, pl.ds(c * BCHUNK, BCHUNK)], osem
            )
        return f

    last_f = lax.fori_loop(0, PPT, pair_body, -1)
    for _ in range(2):
        pltpu.make_async_copy(
            res0, out_hbm.at[last_f, 0, pl.ds(0, BCHUNK)], osem
        ).wait()


def kernel(x_cat, tables):
    x_t = x_cat.T
    tab_t = tables.transpose(0, 2, 1)
    mesh = plsc.VectorSubcoreMesh(core_axis_name="c", subcore_axis_name="s")
    run = functools.partial(
        pl.kernel,
        out_type=jax.ShapeDtypeStruct((N_FIELDS, HIDDEN, BATCH), jnp.float32),
        mesh=mesh,
        compiler_params=pltpu.CompilerParams(needs_layout_passes=False),
        scratch_types=[
            pltpu.VMEM((VOCAB,), jnp.float32),
            pltpu.VMEM((BATCH,), jnp.int32),
            pltpu.VMEM((BCHUNK,), jnp.float32),
            pltpu.VMEM((BCHUNK,), jnp.float32),
            pltpu.SemaphoreType.DMA,
            pltpu.SemaphoreType.DMA,
        ],
    )(_body)
    out_t = run(x_t, tab_t)
    return out_t.transpose(2, 0, 1)
